# baseline (speedup 1.0000x reference)
import jax
import jax.numpy as jnp
from jax.experimental import pallas as pl

_VOCAB = 1000
_ROWS_PER_BLOCK = 512


def _onehot_block(idx_ref, out_ref):
    idx = idx_ref[...]
    iota = jax.lax.broadcasted_iota(jnp.int32, out_ref.shape, 1)
    out_ref[...] = (iota == idx).astype(jnp.float32)


def kernel(inputs):
    b, w = inputs.shape
    n = b * w
    idx = inputs.reshape(n, 1).astype(jnp.int32)
    grid = n // _ROWS_PER_BLOCK
    out = pl.pallas_call(
        _onehot_block,
        grid=(grid,),
        in_specs=[pl.BlockSpec((_ROWS_PER_BLOCK, 1), lambda i: (i, 0))],
        out_specs=pl.BlockSpec((_ROWS_PER_BLOCK, _VOCAB), lambda i: (i, 0)),
        out_shape=jax.ShapeDtypeStruct((n, _VOCAB), jnp.float32),
    )(idx)
    return out.reshape(b, w, _VOCAB)


# speedup vs baseline: 1.5546x; 1.5546x over previous
import jax
import jax.numpy as jnp
from jax.experimental import pallas as pl

_VOCAB = 1000
_BATCH_BLOCK = 32


def _onehot_block(idx_ref, out_ref):
    idx = idx_ref[...]
    iota = jax.lax.broadcasted_iota(jnp.int32, out_ref.shape, 2)
    out_ref[...] = (iota == idx[:, :, None]).astype(jnp.float32)


def kernel(inputs):
    b, w = inputs.shape
    idx = inputs.astype(jnp.int32)
    grid = b // _BATCH_BLOCK
    return pl.pallas_call(
        _onehot_block,
        grid=(grid,),
        in_specs=[pl.BlockSpec((_BATCH_BLOCK, w), lambda i: (i, 0))],
        out_specs=pl.BlockSpec((_BATCH_BLOCK, w, _VOCAB), lambda i: (i, 0, 0)),
        out_shape=jax.ShapeDtypeStruct((b, w, _VOCAB), jnp.float32),
    )(idx)


# speedup vs baseline: 1.5570x; 1.0015x over previous
import jax
import jax.numpy as jnp
from jax.experimental import pallas as pl
from jax.experimental.pallas import tpu as pltpu

_VOCAB = 1000
_B = 32
_K = 4


def _onehot_block(idx_ref, out_ref, vbuf, sems):
    i = pl.program_id(0)
    n = pl.num_programs(0)
    slot = jax.lax.rem(i, _K)

    @pl.when(i >= _K)
    def _wait_prev():
        pltpu.make_async_copy(
            vbuf.at[slot], out_ref.at[pl.ds((i - _K) * _B, _B)], sems.at[slot]
        ).wait()

    idx = idx_ref[...]
    iota = jax.lax.broadcasted_iota(jnp.int32, vbuf.shape[1:], 2)
    vbuf[slot] = (iota == idx[:, :, None]).astype(jnp.float32)
    pltpu.make_async_copy(
        vbuf.at[slot], out_ref.at[pl.ds(i * _B, _B)], sems.at[slot]
    ).start()

    @pl.when(i == n - 1)
    def _drain():
        for j in range(_K):
            pltpu.make_async_copy(
                vbuf.at[j], out_ref.at[pl.ds(0, _B)], sems.at[j]
            ).wait()


def kernel(inputs):
    b, w = inputs.shape
    idx = inputs.astype(jnp.int32)
    grid = b // _B
    return pl.pallas_call(
        _onehot_block,
        grid=(grid,),
        in_specs=[pl.BlockSpec((_B, w), lambda i: (i, 0))],
        out_specs=pl.BlockSpec(memory_space=pl.ANY),
        out_shape=jax.ShapeDtypeStruct((b, w, _VOCAB), jnp.float32),
        scratch_shapes=[
            pltpu.VMEM((_K, _B, w, _VOCAB), jnp.float32),
            pltpu.SemaphoreType.DMA((_K,)),
        ],
    )(idx)
